# baseline (speedup 1.0000x reference)
import functools

import jax
import jax.numpy as jnp
from jax.experimental import pallas as pl
from jax.experimental.pallas import tpu as pltpu
from jax.experimental.pallas import tpu_sc as plsc

_NUM_OPS = 29
_D = 768
_DFF = 256
_H = 12
_HD = 64
_B = 2
_S = 512
_BS = _B * _S
_CARRY_ITERS = 7
_DIV_ITERS = 16
_INV_SQRT_HD = 0.125


def _route_sc_body(x_hbm, out_hbm, v0_v, v1_v, idx_v):
    first = (jax.lax.axis_index("c") == 0) & (jax.lax.axis_index("s") == 0)

    @pl.when(first)
    def _():
        pltpu.sync_copy(x_hbm.at[pl.ds(0, 16)], v0_v)
        pltpu.sync_copy(x_hbm.at[pl.ds(16, 16)], v1_v)
        v0 = v0_v[...]
        lane = jax.lax.iota(jnp.int32, 16)
        v1 = jnp.where(lane + 16 < _NUM_OPS, v1_v[...], jnp.float32(-3e38))
        m = jnp.maximum(v0, v1)
        for sh in (8, 4, 2, 1):
            m = jnp.maximum(m, m.at[lane ^ sh].get(mode="promise_in_bounds"))
        cand = jnp.minimum(jnp.where(v0 == m, lane, 127),
                           jnp.where(v1 == m, lane + 16, 127))
        for sh in (8, 4, 2, 1):
            cand = jnp.minimum(
                cand, cand.at[lane ^ sh].get(mode="promise_in_bounds"))
        idx_v[...] = cand
        pltpu.sync_copy(idx_v, out_hbm)


_route_sc = pl.kernel(
    _route_sc_body,
    mesh=plsc.VectorSubcoreMesh(core_axis_name="c", subcore_axis_name="s"),
    out_type=jax.ShapeDtypeStruct((16,), jnp.int32),
    scratch_types=[
        pltpu.VMEM((16,), jnp.float32),
        pltpu.VMEM((16,), jnp.float32),
        pltpu.VMEM((16,), jnp.int32),
    ],
)


def _route(x):
    xs = jax.lax.slice(x.reshape(_B * _S * _D), (0,), (32,))
    return _route_sc(xs)[:1]


def _expert_body(a_ref, x_ref, w1_ref, w2_ref, o_ref):
    xb = x_ref[...]
    h = jnp.maximum(jnp.dot(xb, w1_ref[0], preferred_element_type=jnp.float32), 0.0)
    o_ref[...] = xb + jnp.dot(h, w2_ref[0], preferred_element_type=jnp.float32)


def _expert(x2, w1_tbl, w2_tbl, active):
    grid_spec = pltpu.PrefetchScalarGridSpec(
        num_scalar_prefetch=1,
        grid=(1,),
        in_specs=[
            pl.BlockSpec((_BS, _D), lambda i, a: (0, 0)),
            pl.BlockSpec((1, _D, _DFF), lambda i, a: (a[0], 0, 0)),
            pl.BlockSpec((1, _DFF, _D), lambda i, a: (a[0], 0, 0)),
        ],
        out_specs=pl.BlockSpec((_BS, _D), lambda i, a: (0, 0)),
    )
    return pl.pallas_call(
        _expert_body,
        grid_spec=grid_spec,
        out_shape=jax.ShapeDtypeStruct((_BS, _D), jnp.float32),
    )(active, x2, w1_tbl, w2_tbl)


def _prep_mha(A):
    Aq = A[0].reshape(_D, _H, _HD).transpose(1, 0, 2)
    Ak = A[1].reshape(_D, _H, _HD).transpose(1, 0, 2)
    Av = A[2].reshape(_D, _H, _HD).transpose(1, 0, 2)
    Ao = A[3].reshape(_H, _HD, _D)
    return Aq, Ak, Av, Ao


def _attn_head(xb, wq, wk, wv, wo):
    q = jnp.dot(xb, wq, preferred_element_type=jnp.float32)
    k = jnp.dot(xb, wk, preferred_element_type=jnp.float32)
    v = jnp.dot(xb, wv, preferred_element_type=jnp.float32)
    s = jnp.dot(q, k.T, preferred_element_type=jnp.float32) * _INV_SQRT_HD
    p = jax.nn.softmax(s, axis=-1)
    o = jnp.dot(p, v, preferred_element_type=jnp.float32)
    return jnp.dot(o, wo, preferred_element_type=jnp.float32)


def _mha_body(x_ref, aq_ref, ak_ref, av_ref, ao_ref, o_ref):
    h = pl.program_id(1)
    xb = x_ref[0]
    contrib = _attn_head(xb, aq_ref[0], ak_ref[0], av_ref[0], ao_ref[0])

    @pl.when(h == 0)
    def _():
        o_ref[0] = xb + contrib

    @pl.when(h != 0)
    def _():
        o_ref[0] = o_ref[0] + contrib


def _mha(x3, A):
    Aq, Ak, Av, Ao = _prep_mha(A)
    return pl.pallas_call(
        _mha_body,
        grid=(_B, _H),
        in_specs=[
            pl.BlockSpec((1, _S, _D), lambda b, h: (b, 0, 0)),
            pl.BlockSpec((1, _D, _HD), lambda b, h: (h, 0, 0)),
            pl.BlockSpec((1, _D, _HD), lambda b, h: (h, 0, 0)),
            pl.BlockSpec((1, _D, _HD), lambda b, h: (h, 0, 0)),
            pl.BlockSpec((1, _HD, _D), lambda b, h: (h, 0, 0)),
        ],
        out_specs=pl.BlockSpec((1, _S, _D), lambda b, h: (b, 0, 0)),
        out_shape=jax.ShapeDtypeStruct((_B, _S, _D), jnp.float32),
    )(x3, Aq, Ak, Av, Ao)


_BF = jnp.bfloat16


def _ffn_res(xa, w1_bf, w2_bf):
    h = jnp.maximum(
        jnp.dot(xa.astype(_BF), w1_bf, preferred_element_type=jnp.float32), 0.0)
    return xa + jnp.dot(h.astype(_BF), w2_bf, preferred_element_type=jnp.float32)


_LOG2E = 1.4426950408889634


def _carry_body(a_ref, x_ref, ac_ref, w1l0_ref, w2l0_ref,
                w1_ref, w2_ref, *rest):
    if len(rest) == 10:
        (w1f_ref, w2f_ref, w1p_ref, w2p_ref,
         o_ref, qkv_ref, oc_ref, vaug_ref, wqkv_ref, wo_ref) = rest
    else:
        o_ref, qkv_ref, oc_ref, vaug_ref, wqkv_ref, wo_ref = rest
        w1f_ref = None
    for i in range(3):
        wqkv_ref[:, i * _D:(i + 1) * _D] = ac_ref[i].astype(_BF)
    wo_ref[...] = ac_ref[3].astype(_BF)
    w1l0 = w1l0_ref[0].astype(_BF)
    w2l0 = w2l0_ref[0].astype(_BF)
    w1 = w1_ref[0].astype(_BF)
    w2 = w2_ref[0].astype(_BF)
    o_ref[...] = _ffn_res(x_ref[...].reshape(_BS, _D), w1l0,
                          w2l0).reshape(_B, _S, _D)

    lane = jax.lax.broadcasted_iota(jnp.int32, (_BS, 2 * _HD), 1)
    ones_pat = jnp.where(lane == _HD, 1.0, 0.0).astype(_BF)
    for h in range(_H):
        vaug_ref[:, h * 2 * _HD:(h + 1) * 2 * _HD] = ones_pat

    def iter_body(_, carry):
        del carry
        xa = o_ref[...].reshape(_BS, _D)
        qkv_ref[...] = jnp.dot(
            xa.astype(_BF), wqkv_ref[...],
            preferred_element_type=jnp.float32).astype(_BF)
        for h in range(_H):
            vaug_ref[:, h * 2 * _HD:h * 2 * _HD + _HD] = (
                qkv_ref[:, 2 * _D + h * _HD:2 * _D + (h + 1) * _HD])
        for b in range(_B):
            for h in range(_H):
                r0 = b * _S
                q = qkv_ref[r0:r0 + _S, h * _HD:(h + 1) * _HD]
                k = qkv_ref[r0:r0 + _S, _D + h * _HD:_D + (h + 1) * _HD]
                s = jax.lax.dot_general(
                    q, k, (((1,), (1,)), ((), ())),
                    preferred_element_type=jnp.float32) * (_INV_SQRT_HD * _LOG2E)
                p = jnp.exp2(s)
                o_aug = jnp.dot(p.astype(_BF),
                                vaug_ref[r0:r0 + _S, h * 2 * _HD:(h + 1) * 2 * _HD],
                                preferred_element_type=jnp.float32)
                o = o_aug[:, :_HD] / o_aug[:, _HD:_HD + 1]
                oc_ref[r0:r0 + _S, h * _HD:(h + 1) * _HD] = o.astype(_BF)
        xm = xa + jnp.dot(oc_ref[...], wo_ref[...],
                          preferred_element_type=jnp.float32)
        o_ref[...] = _ffn_res(xm, w1, w2).reshape(_B, _S, _D)
        return 0

    jax.lax.fori_loop(0, _CARRY_ITERS, iter_body, 0, unroll=False)

    if w1f_ref is not None:
        xa = o_ref[...].reshape(_BS, _D)
        xa = _ffn_res(xa, w1f_ref[0].astype(_BF), w2f_ref[0].astype(_BF))
        xa = _ffn_res(xa, w1p_ref[0].astype(_BF), w2p_ref[0].astype(_BF))
        o_ref[...] = xa.reshape(_B, _S, _D)


def _carry(x3, A_carry, w1l0_tbl, w2l0_tbl, w1_tbl, w2_tbl, active, tail=None):
    row_spec = lambda i, a: (a[0], 0, 0)
    w_specs = [
        pl.BlockSpec((1, _D, _DFF), row_spec),
        pl.BlockSpec((1, _DFF, _D), row_spec),
    ]
    in_specs = [
        pl.BlockSpec((_B, _S, _D), lambda i, a: (0, 0, 0)),
        pl.BlockSpec((4, _D, _D), lambda i, a: (0, 0, 0)),
    ] + w_specs * (4 if tail is not None else 2)
    args = [active, x3, A_carry, w1l0_tbl, w2l0_tbl, w1_tbl, w2_tbl]
    if tail is not None:
        args += list(tail)
    grid_spec = pltpu.PrefetchScalarGridSpec(
        num_scalar_prefetch=1,
        grid=(1,),
        in_specs=in_specs,
        out_specs=pl.BlockSpec((_B, _S, _D), lambda i, a: (0, 0, 0)),
        scratch_shapes=[
            pltpu.VMEM((_BS, 3 * _D), _BF),
            pltpu.VMEM((_BS, _D), _BF),
            pltpu.VMEM((_BS, _H * 2 * _HD), _BF),
            pltpu.VMEM((_D, 3 * _D), _BF),
            pltpu.VMEM((_D, _D), _BF),
        ],
    )
    return pl.pallas_call(
        _carry_body,
        grid_spec=grid_spec,
        out_shape=jax.ShapeDtypeStruct((_B, _S, _D), jnp.float32),
    )(*args)


def _stack_body(x_ref, w1_ref, w2_ref, o_ref):
    i = pl.program_id(0)

    @pl.when(i == 0)
    def _():
        o_ref[...] = x_ref[...]

    cur = o_ref[...]
    h = jnp.maximum(jnp.dot(cur, w1_ref[0], preferred_element_type=jnp.float32), 0.0)
    o_ref[...] = cur + jnp.dot(h, w2_ref[0], preferred_element_type=jnp.float32)


def _stack(x2, w1_tbl, w2_tbl):
    return pl.pallas_call(
        _stack_body,
        grid=(_DIV_ITERS,),
        in_specs=[
            pl.BlockSpec((_BS, _D), lambda i: (0, 0)),
            pl.BlockSpec((1, _D, _DFF), lambda i: (i, 0, 0)),
            pl.BlockSpec((1, _DFF, _D), lambda i: (i, 0, 0)),
        ],
        out_specs=pl.BlockSpec((_BS, _D), lambda i: (0, 0)),
        out_shape=jax.ShapeDtypeStruct((_BS, _D), jnp.float32),
    )(x2, w1_tbl, w2_tbl)


def kernel(x, W1_l0, W2_l0, A_carry, W1_carry, W2_carry, W1_div, W2_div,
           W1_mod, W2_mod, A_shl, A_shr, A_eq, A_ne, A_cmp,
           W1_final, W2_final, W1_post, W2_post):
    active = _route(x)
    a0 = active[0]

    def carry_fused_path(t):
        return _carry(t, A_carry, W1_l0, W2_l0, W1_carry, W2_carry, active,
                      tail=(W1_final, W2_final, W1_post, W2_post))

    def carry_cmp_path(t):
        u = _carry(t, A_carry, W1_l0, W2_l0, W1_carry, W2_carry, active)
        u = _expert(u.reshape(_BS, _D), W1_final, W2_final, active)
        u = _mha(u.reshape(_B, _S, _D), A_cmp)
        u = _expert(u.reshape(_BS, _D), W1_post, W2_post, active)
        return u.reshape(_B, _S, _D)

    def other_path(t):
        u = _expert(t.reshape(_BS, _D), W1_l0, W2_l0, active)
        u3 = u.reshape(_B, _S, _D)
        u3 = jax.lax.cond(
            a0 == 3,
            lambda v: _stack(v.reshape(_BS, _D), W1_div,
                             W2_div).reshape(_B, _S, _D),
            lambda v: v, u3)
        u3 = jax.lax.cond(
            a0 == 4,
            lambda v: _stack(v.reshape(_BS, _D), W1_mod,
                             W2_mod).reshape(_B, _S, _D),
            lambda v: v, u3)
        u3 = jax.lax.cond(
            a0 == 14, lambda v: _mha(v, A_shl),
            lambda v: jax.lax.cond(a0 == 15, lambda w: _mha(w, A_shr),
                                   lambda w: w, v),
            u3)
        u3 = jax.lax.cond(
            a0 == 8, lambda v: _mha(v, A_eq),
            lambda v: jax.lax.cond(a0 == 9, lambda w: _mha(w, A_ne),
                                   lambda w: w, v),
            u3)
        u = _expert(u3.reshape(_BS, _D), W1_final, W2_final, active)
        u = _expert(u, W1_post, W2_post, active)
        return u.reshape(_B, _S, _D)

    return jax.lax.cond(
        a0 <= 2, carry_fused_path,
        lambda t: jax.lax.cond((a0 >= 10) & (a0 <= 13),
                               carry_cmp_path, other_path, t),
        x)


# speedup vs baseline: 1.1590x; 1.1590x over previous
import functools

import jax
import jax.numpy as jnp
from jax.experimental import pallas as pl
from jax.experimental.pallas import tpu as pltpu

_NUM_OPS = 29
_D = 768
_DFF = 256
_H = 12
_HD = 64
_B = 2
_S = 512
_BS = _B * _S
_CARRY_ITERS = 7
_DIV_ITERS = 16
_INV_SQRT_HD = 0.125


def _route_body(x_ref, o_ref):
    v = x_ref[0]
    lane = jax.lax.broadcasted_iota(jnp.int32, (1, 128), 1)
    m = jnp.where(lane < _NUM_OPS, v, -jnp.inf)
    o_ref[0] = jnp.argmax(m).astype(jnp.int32)


def _route(x):
    xs = jax.lax.slice(x, (0, 0, 0), (1, 1, 128))
    return pl.pallas_call(
        _route_body,
        out_shape=jax.ShapeDtypeStruct((1,), jnp.int32),
        in_specs=[pl.BlockSpec((1, 1, 128), lambda: (0, 0, 0))],
        out_specs=pl.BlockSpec(memory_space=pltpu.SMEM),
    )(xs)


def _expert_body(a_ref, x_ref, w1_ref, w2_ref, o_ref):
    xb = x_ref[...]
    h = jnp.maximum(jnp.dot(xb, w1_ref[0], preferred_element_type=jnp.float32), 0.0)
    o_ref[...] = xb + jnp.dot(h, w2_ref[0], preferred_element_type=jnp.float32)


def _expert(x2, w1_tbl, w2_tbl, active):
    grid_spec = pltpu.PrefetchScalarGridSpec(
        num_scalar_prefetch=1,
        grid=(1,),
        in_specs=[
            pl.BlockSpec((_BS, _D), lambda i, a: (0, 0)),
            pl.BlockSpec((1, _D, _DFF), lambda i, a: (a[0], 0, 0)),
            pl.BlockSpec((1, _DFF, _D), lambda i, a: (a[0], 0, 0)),
        ],
        out_specs=pl.BlockSpec((_BS, _D), lambda i, a: (0, 0)),
    )
    return pl.pallas_call(
        _expert_body,
        grid_spec=grid_spec,
        out_shape=jax.ShapeDtypeStruct((_BS, _D), jnp.float32),
    )(active, x2, w1_tbl, w2_tbl)


def _prep_mha(A):
    Aq = A[0].reshape(_D, _H, _HD).transpose(1, 0, 2)
    Ak = A[1].reshape(_D, _H, _HD).transpose(1, 0, 2)
    Av = A[2].reshape(_D, _H, _HD).transpose(1, 0, 2)
    Ao = A[3].reshape(_H, _HD, _D)
    return Aq, Ak, Av, Ao


def _attn_head(xb, wq, wk, wv, wo):
    q = jnp.dot(xb, wq, preferred_element_type=jnp.float32)
    k = jnp.dot(xb, wk, preferred_element_type=jnp.float32)
    v = jnp.dot(xb, wv, preferred_element_type=jnp.float32)
    s = jnp.dot(q, k.T, preferred_element_type=jnp.float32) * _INV_SQRT_HD
    p = jax.nn.softmax(s, axis=-1)
    o = jnp.dot(p, v, preferred_element_type=jnp.float32)
    return jnp.dot(o, wo, preferred_element_type=jnp.float32)


def _mha_body(x_ref, aq_ref, ak_ref, av_ref, ao_ref, o_ref):
    h = pl.program_id(1)
    xb = x_ref[0]
    contrib = _attn_head(xb, aq_ref[0], ak_ref[0], av_ref[0], ao_ref[0])

    @pl.when(h == 0)
    def _():
        o_ref[0] = xb + contrib

    @pl.when(h != 0)
    def _():
        o_ref[0] = o_ref[0] + contrib


def _mha(x3, A):
    Aq, Ak, Av, Ao = _prep_mha(A)
    return pl.pallas_call(
        _mha_body,
        grid=(_B, _H),
        in_specs=[
            pl.BlockSpec((1, _S, _D), lambda b, h: (b, 0, 0)),
            pl.BlockSpec((1, _D, _HD), lambda b, h: (h, 0, 0)),
            pl.BlockSpec((1, _D, _HD), lambda b, h: (h, 0, 0)),
            pl.BlockSpec((1, _D, _HD), lambda b, h: (h, 0, 0)),
            pl.BlockSpec((1, _HD, _D), lambda b, h: (h, 0, 0)),
        ],
        out_specs=pl.BlockSpec((1, _S, _D), lambda b, h: (b, 0, 0)),
        out_shape=jax.ShapeDtypeStruct((_B, _S, _D), jnp.float32),
    )(x3, Aq, Ak, Av, Ao)


_BF = jnp.bfloat16


def _ffn_res(xa, w1_bf, w2_bf):
    h = jnp.maximum(
        jnp.dot(xa.astype(_BF), w1_bf, preferred_element_type=jnp.float32), 0.0)
    return xa + jnp.dot(h.astype(_BF), w2_bf, preferred_element_type=jnp.float32)


_LOG2E = 1.4426950408889634


def _carry_body(a_ref, x_ref, ac_ref, w1l0_ref, w2l0_ref,
                w1_ref, w2_ref, *rest):
    if len(rest) == 10:
        (w1f_ref, w2f_ref, w1p_ref, w2p_ref,
         o_ref, qkv_ref, oc_ref, vaug_ref, wqkv_ref, wo_ref) = rest
    else:
        o_ref, qkv_ref, oc_ref, vaug_ref, wqkv_ref, wo_ref = rest
        w1f_ref = None
    for i in range(3):
        wqkv_ref[:, i * _D:(i + 1) * _D] = ac_ref[i].astype(_BF)
    wo_ref[...] = ac_ref[3].astype(_BF)
    w1l0 = w1l0_ref[0].astype(_BF)
    w2l0 = w2l0_ref[0].astype(_BF)
    w1 = w1_ref[0].astype(_BF)
    w2 = w2_ref[0].astype(_BF)
    o_ref[...] = _ffn_res(x_ref[...].reshape(_BS, _D), w1l0,
                          w2l0).reshape(_B, _S, _D)

    lane = jax.lax.broadcasted_iota(jnp.int32, (_BS, 2 * _HD), 1)
    ones_pat = jnp.where(lane == _HD, 1.0, 0.0).astype(_BF)
    for h in range(_H):
        vaug_ref[:, h * 2 * _HD:(h + 1) * 2 * _HD] = ones_pat

    def iter_body(_, carry):
        del carry
        xa = o_ref[...].reshape(_BS, _D)
        qkv_ref[...] = jnp.dot(
            xa.astype(_BF), wqkv_ref[...],
            preferred_element_type=jnp.float32).astype(_BF)
        for h in range(_H):
            vaug_ref[:, h * 2 * _HD:h * 2 * _HD + _HD] = (
                qkv_ref[:, 2 * _D + h * _HD:2 * _D + (h + 1) * _HD])
        for b in range(_B):
            for h in range(_H):
                r0 = b * _S
                q = qkv_ref[r0:r0 + _S, h * _HD:(h + 1) * _HD]
                k = qkv_ref[r0:r0 + _S, _D + h * _HD:_D + (h + 1) * _HD]
                s = jax.lax.dot_general(
                    q, k, (((1,), (1,)), ((), ())),
                    preferred_element_type=jnp.float32) * (_INV_SQRT_HD * _LOG2E)
                p = jnp.exp2(s)
                o_aug = jnp.dot(p.astype(_BF),
                                vaug_ref[r0:r0 + _S, h * 2 * _HD:(h + 1) * 2 * _HD],
                                preferred_element_type=jnp.float32)
                o = o_aug[:, :_HD] / o_aug[:, _HD:_HD + 1]
                oc_ref[r0:r0 + _S, h * _HD:(h + 1) * _HD] = o.astype(_BF)
        xm = xa + jnp.dot(oc_ref[...], wo_ref[...],
                          preferred_element_type=jnp.float32)
        o_ref[...] = _ffn_res(xm, w1, w2).reshape(_B, _S, _D)
        return 0

    jax.lax.fori_loop(0, _CARRY_ITERS, iter_body, 0, unroll=False)

    if w1f_ref is not None:
        xa = o_ref[...].reshape(_BS, _D)
        xa = _ffn_res(xa, w1f_ref[0].astype(_BF), w2f_ref[0].astype(_BF))
        xa = _ffn_res(xa, w1p_ref[0].astype(_BF), w2p_ref[0].astype(_BF))
        o_ref[...] = xa.reshape(_B, _S, _D)


def _carry(x3, A_carry, w1l0_tbl, w2l0_tbl, w1_tbl, w2_tbl, active, tail=None):
    row_spec = lambda i, a: (a[0], 0, 0)
    w_specs = [
        pl.BlockSpec((1, _D, _DFF), row_spec),
        pl.BlockSpec((1, _DFF, _D), row_spec),
    ]
    in_specs = [
        pl.BlockSpec((_B, _S, _D), lambda i, a: (0, 0, 0)),
        pl.BlockSpec((4, _D, _D), lambda i, a: (0, 0, 0)),
    ] + w_specs * (4 if tail is not None else 2)
    args = [active, x3, A_carry, w1l0_tbl, w2l0_tbl, w1_tbl, w2_tbl]
    if tail is not None:
        args += list(tail)
    grid_spec = pltpu.PrefetchScalarGridSpec(
        num_scalar_prefetch=1,
        grid=(1,),
        in_specs=in_specs,
        out_specs=pl.BlockSpec((_B, _S, _D), lambda i, a: (0, 0, 0)),
        scratch_shapes=[
            pltpu.VMEM((_BS, 3 * _D), _BF),
            pltpu.VMEM((_BS, _D), _BF),
            pltpu.VMEM((_BS, _H * 2 * _HD), _BF),
            pltpu.VMEM((_D, 3 * _D), _BF),
            pltpu.VMEM((_D, _D), _BF),
        ],
    )
    return pl.pallas_call(
        _carry_body,
        grid_spec=grid_spec,
        out_shape=jax.ShapeDtypeStruct((_B, _S, _D), jnp.float32),
    )(*args)


def _stack_body(x_ref, w1_ref, w2_ref, o_ref):
    i = pl.program_id(0)

    @pl.when(i == 0)
    def _():
        o_ref[...] = x_ref[...]

    cur = o_ref[...]
    h = jnp.maximum(jnp.dot(cur, w1_ref[0], preferred_element_type=jnp.float32), 0.0)
    o_ref[...] = cur + jnp.dot(h, w2_ref[0], preferred_element_type=jnp.float32)


def _stack(x2, w1_tbl, w2_tbl):
    return pl.pallas_call(
        _stack_body,
        grid=(_DIV_ITERS,),
        in_specs=[
            pl.BlockSpec((_BS, _D), lambda i: (0, 0)),
            pl.BlockSpec((1, _D, _DFF), lambda i: (i, 0, 0)),
            pl.BlockSpec((1, _DFF, _D), lambda i: (i, 0, 0)),
        ],
        out_specs=pl.BlockSpec((_BS, _D), lambda i: (0, 0)),
        out_shape=jax.ShapeDtypeStruct((_BS, _D), jnp.float32),
    )(x2, w1_tbl, w2_tbl)


def kernel(x, W1_l0, W2_l0, A_carry, W1_carry, W2_carry, W1_div, W2_div,
           W1_mod, W2_mod, A_shl, A_shr, A_eq, A_ne, A_cmp,
           W1_final, W2_final, W1_post, W2_post):
    active = _route(x)
    a0 = active[0]

    def carry_fused_path(t):
        return _carry(t, A_carry, W1_l0, W2_l0, W1_carry, W2_carry, active,
                      tail=(W1_final, W2_final, W1_post, W2_post))

    def carry_cmp_path(t):
        u = _carry(t, A_carry, W1_l0, W2_l0, W1_carry, W2_carry, active)
        u = _expert(u.reshape(_BS, _D), W1_final, W2_final, active)
        u = _mha(u.reshape(_B, _S, _D), A_cmp)
        u = _expert(u.reshape(_BS, _D), W1_post, W2_post, active)
        return u.reshape(_B, _S, _D)

    def other_path(t):
        u = _expert(t.reshape(_BS, _D), W1_l0, W2_l0, active)
        u3 = u.reshape(_B, _S, _D)
        u3 = jax.lax.cond(
            a0 == 3,
            lambda v: _stack(v.reshape(_BS, _D), W1_div,
                             W2_div).reshape(_B, _S, _D),
            lambda v: v, u3)
        u3 = jax.lax.cond(
            a0 == 4,
            lambda v: _stack(v.reshape(_BS, _D), W1_mod,
                             W2_mod).reshape(_B, _S, _D),
            lambda v: v, u3)
        u3 = jax.lax.cond(
            a0 == 14, lambda v: _mha(v, A_shl),
            lambda v: jax.lax.cond(a0 == 15, lambda w: _mha(w, A_shr),
                                   lambda w: w, v),
            u3)
        u3 = jax.lax.cond(
            a0 == 8, lambda v: _mha(v, A_eq),
            lambda v: jax.lax.cond(a0 == 9, lambda w: _mha(w, A_ne),
                                   lambda w: w, v),
            u3)
        u = _expert(u3.reshape(_BS, _D), W1_final, W2_final, active)
        u = _expert(u, W1_post, W2_post, active)
        return u.reshape(_B, _S, _D)

    return jax.lax.cond(
        a0 <= 2, carry_fused_path,
        lambda t: jax.lax.cond((a0 >= 10) & (a0 <= 13),
                               carry_cmp_path, other_path, t),
        x)


# speedup vs baseline: 1.1635x; 1.0039x over previous
import functools

import jax
import jax.numpy as jnp
from jax.experimental import pallas as pl
from jax.experimental.pallas import tpu as pltpu

_NUM_OPS = 29
_D = 768
_DFF = 256
_H = 12
_HD = 64
_B = 2
_S = 512
_BS = _B * _S
_CARRY_ITERS = 7
_DIV_ITERS = 16
_INV_SQRT_HD = 0.125


def _route_body(x_ref, o_ref):
    v = x_ref[0]
    lane = jax.lax.broadcasted_iota(jnp.int32, (1, 128), 1)
    m = jnp.where(lane < _NUM_OPS, v, -jnp.inf)
    o_ref[0] = jnp.argmax(m).astype(jnp.int32)


def _route(x):
    xs = jax.lax.slice(x, (0, 0, 0), (1, 1, 128))
    return pl.pallas_call(
        _route_body,
        out_shape=jax.ShapeDtypeStruct((1,), jnp.int32),
        in_specs=[pl.BlockSpec((1, 1, 128), lambda: (0, 0, 0))],
        out_specs=pl.BlockSpec(memory_space=pltpu.SMEM),
    )(xs)


def _expert_body(a_ref, x_ref, w1_ref, w2_ref, o_ref):
    xb = x_ref[...]
    h = jnp.maximum(jnp.dot(xb, w1_ref[0], preferred_element_type=jnp.float32), 0.0)
    o_ref[...] = xb + jnp.dot(h, w2_ref[0], preferred_element_type=jnp.float32)


def _expert(x2, w1_tbl, w2_tbl, active):
    grid_spec = pltpu.PrefetchScalarGridSpec(
        num_scalar_prefetch=1,
        grid=(1,),
        in_specs=[
            pl.BlockSpec((_BS, _D), lambda i, a: (0, 0)),
            pl.BlockSpec((1, _D, _DFF), lambda i, a: (a[0], 0, 0)),
            pl.BlockSpec((1, _DFF, _D), lambda i, a: (a[0], 0, 0)),
        ],
        out_specs=pl.BlockSpec((_BS, _D), lambda i, a: (0, 0)),
    )
    return pl.pallas_call(
        _expert_body,
        grid_spec=grid_spec,
        out_shape=jax.ShapeDtypeStruct((_BS, _D), jnp.float32),
    )(active, x2, w1_tbl, w2_tbl)


def _prep_mha(A):
    Aq = A[0].reshape(_D, _H, _HD).transpose(1, 0, 2)
    Ak = A[1].reshape(_D, _H, _HD).transpose(1, 0, 2)
    Av = A[2].reshape(_D, _H, _HD).transpose(1, 0, 2)
    Ao = A[3].reshape(_H, _HD, _D)
    return Aq, Ak, Av, Ao


def _attn_head(xb, wq, wk, wv, wo):
    q = jnp.dot(xb, wq, preferred_element_type=jnp.float32)
    k = jnp.dot(xb, wk, preferred_element_type=jnp.float32)
    v = jnp.dot(xb, wv, preferred_element_type=jnp.float32)
    s = jnp.dot(q, k.T, preferred_element_type=jnp.float32) * _INV_SQRT_HD
    p = jax.nn.softmax(s, axis=-1)
    o = jnp.dot(p, v, preferred_element_type=jnp.float32)
    return jnp.dot(o, wo, preferred_element_type=jnp.float32)


def _mha_body(x_ref, aq_ref, ak_ref, av_ref, ao_ref, o_ref):
    h = pl.program_id(1)
    xb = x_ref[0]
    contrib = _attn_head(xb, aq_ref[0], ak_ref[0], av_ref[0], ao_ref[0])

    @pl.when(h == 0)
    def _():
        o_ref[0] = xb + contrib

    @pl.when(h != 0)
    def _():
        o_ref[0] = o_ref[0] + contrib


def _mha(x3, A):
    Aq, Ak, Av, Ao = _prep_mha(A)
    return pl.pallas_call(
        _mha_body,
        grid=(_B, _H),
        in_specs=[
            pl.BlockSpec((1, _S, _D), lambda b, h: (b, 0, 0)),
            pl.BlockSpec((1, _D, _HD), lambda b, h: (h, 0, 0)),
            pl.BlockSpec((1, _D, _HD), lambda b, h: (h, 0, 0)),
            pl.BlockSpec((1, _D, _HD), lambda b, h: (h, 0, 0)),
            pl.BlockSpec((1, _HD, _D), lambda b, h: (h, 0, 0)),
        ],
        out_specs=pl.BlockSpec((1, _S, _D), lambda b, h: (b, 0, 0)),
        out_shape=jax.ShapeDtypeStruct((_B, _S, _D), jnp.float32),
    )(x3, Aq, Ak, Av, Ao)


_BF = jnp.bfloat16


def _ffn_res(xa, w1_bf, w2_bf):
    h = jnp.maximum(
        jnp.dot(xa.astype(_BF), w1_bf, preferred_element_type=jnp.float32), 0.0)
    return xa + jnp.dot(h.astype(_BF), w2_bf, preferred_element_type=jnp.float32)


_LOG2E = 1.4426950408889634


def _carry_body(a_ref, x_ref, ac_ref, w1l0_ref, w2l0_ref,
                w1_ref, w2_ref, *rest):
    if len(rest) == 11:
        (w1f_ref, w2f_ref, w1p_ref, w2p_ref,
         o_ref, qkv_ref, oc_ref, vaug_ref, wqkv_ref, wo_ref, kt_ref) = rest
    else:
        (o_ref, qkv_ref, oc_ref, vaug_ref, wqkv_ref, wo_ref, kt_ref) = rest
        w1f_ref = None
    for pos, i in enumerate((0, 2, 1)):
        wqkv_ref[:, pos * _D:(pos + 1) * _D] = ac_ref[i].astype(_BF)
    wo_ref[...] = ac_ref[3].astype(_BF)
    w1l0 = w1l0_ref[0].astype(_BF)
    w2l0 = w2l0_ref[0].astype(_BF)
    w1 = w1_ref[0].astype(_BF)
    w2 = w2_ref[0].astype(_BF)
    o_ref[...] = _ffn_res(x_ref[...].reshape(_BS, _D), w1l0,
                          w2l0).reshape(_B, _S, _D)

    lane = jax.lax.broadcasted_iota(jnp.int32, (_BS, 2 * _HD), 1)
    ones_pat = jnp.where(lane == _HD, 1.0, 0.0).astype(_BF)
    for h in range(_H):
        vaug_ref[:, h * 2 * _HD:(h + 1) * 2 * _HD] = ones_pat

    def iter_body(_, carry):
        del carry
        xa = o_ref[...].reshape(_BS, _D)
        xa_bf = xa.astype(_BF)
        qkv_ref[...] = jnp.dot(
            xa_bf, wqkv_ref[:, :2 * _D],
            preferred_element_type=jnp.float32).astype(_BF)
        kt_ref[...] = jax.lax.dot_general(
            wqkv_ref[:, 2 * _D:], xa_bf, (((0,), (1,)), ((), ())),
            preferred_element_type=jnp.float32).astype(_BF)
        for h in range(_H):
            vaug_ref[:, h * 2 * _HD:h * 2 * _HD + _HD] = (
                qkv_ref[:, _D + h * _HD:_D + (h + 1) * _HD])
        for b in range(_B):
            for h in range(_H):
                r0 = b * _S
                q = qkv_ref[r0:r0 + _S, h * _HD:(h + 1) * _HD]
                kt = kt_ref[h * _HD:(h + 1) * _HD, r0:r0 + _S]
                s = jnp.dot(
                    q, kt,
                    preferred_element_type=jnp.float32) * (_INV_SQRT_HD * _LOG2E)
                p = jnp.exp2(s)
                o_aug = jnp.dot(p.astype(_BF),
                                vaug_ref[r0:r0 + _S, h * 2 * _HD:(h + 1) * 2 * _HD],
                                preferred_element_type=jnp.float32)
                o = o_aug[:, :_HD] / o_aug[:, _HD:_HD + 1]
                oc_ref[r0:r0 + _S, h * _HD:(h + 1) * _HD] = o.astype(_BF)
        xm = xa + jnp.dot(oc_ref[...], wo_ref[...],
                          preferred_element_type=jnp.float32)
        o_ref[...] = _ffn_res(xm, w1, w2).reshape(_B, _S, _D)
        return 0

    jax.lax.fori_loop(0, _CARRY_ITERS, iter_body, 0, unroll=False)

    if w1f_ref is not None:
        xa = o_ref[...].reshape(_BS, _D)
        xa = _ffn_res(xa, w1f_ref[0].astype(_BF), w2f_ref[0].astype(_BF))
        xa = _ffn_res(xa, w1p_ref[0].astype(_BF), w2p_ref[0].astype(_BF))
        o_ref[...] = xa.reshape(_B, _S, _D)


def _carry(x3, A_carry, w1l0_tbl, w2l0_tbl, w1_tbl, w2_tbl, active, tail=None):
    row_spec = lambda i, a: (a[0], 0, 0)
    w_specs = [
        pl.BlockSpec((1, _D, _DFF), row_spec),
        pl.BlockSpec((1, _DFF, _D), row_spec),
    ]
    in_specs = [
        pl.BlockSpec((_B, _S, _D), lambda i, a: (0, 0, 0)),
        pl.BlockSpec((4, _D, _D), lambda i, a: (0, 0, 0)),
    ] + w_specs * (4 if tail is not None else 2)
    args = [active, x3, A_carry, w1l0_tbl, w2l0_tbl, w1_tbl, w2_tbl]
    if tail is not None:
        args += list(tail)
    grid_spec = pltpu.PrefetchScalarGridSpec(
        num_scalar_prefetch=1,
        grid=(1,),
        in_specs=in_specs,
        out_specs=pl.BlockSpec((_B, _S, _D), lambda i, a: (0, 0, 0)),
        scratch_shapes=[
            pltpu.VMEM((_BS, 2 * _D), _BF),
            pltpu.VMEM((_BS, _D), _BF),
            pltpu.VMEM((_BS, _H * 2 * _HD), _BF),
            pltpu.VMEM((_D, 3 * _D), _BF),
            pltpu.VMEM((_D, _D), _BF),
            pltpu.VMEM((_D, _BS), _BF),
        ],
    )
    return pl.pallas_call(
        _carry_body,
        grid_spec=grid_spec,
        out_shape=jax.ShapeDtypeStruct((_B, _S, _D), jnp.float32),
    )(*args)


def _stack_body(x_ref, w1_ref, w2_ref, o_ref):
    i = pl.program_id(0)

    @pl.when(i == 0)
    def _():
        o_ref[...] = x_ref[...]

    cur = o_ref[...]
    h = jnp.maximum(jnp.dot(cur, w1_ref[0], preferred_element_type=jnp.float32), 0.0)
    o_ref[...] = cur + jnp.dot(h, w2_ref[0], preferred_element_type=jnp.float32)


def _stack(x2, w1_tbl, w2_tbl):
    return pl.pallas_call(
        _stack_body,
        grid=(_DIV_ITERS,),
        in_specs=[
            pl.BlockSpec((_BS, _D), lambda i: (0, 0)),
            pl.BlockSpec((1, _D, _DFF), lambda i: (i, 0, 0)),
            pl.BlockSpec((1, _DFF, _D), lambda i: (i, 0, 0)),
        ],
        out_specs=pl.BlockSpec((_BS, _D), lambda i: (0, 0)),
        out_shape=jax.ShapeDtypeStruct((_BS, _D), jnp.float32),
    )(x2, w1_tbl, w2_tbl)


def kernel(x, W1_l0, W2_l0, A_carry, W1_carry, W2_carry, W1_div, W2_div,
           W1_mod, W2_mod, A_shl, A_shr, A_eq, A_ne, A_cmp,
           W1_final, W2_final, W1_post, W2_post):
    active = _route(x)
    a0 = active[0]

    def carry_fused_path(t):
        return _carry(t, A_carry, W1_l0, W2_l0, W1_carry, W2_carry, active,
                      tail=(W1_final, W2_final, W1_post, W2_post))

    def carry_cmp_path(t):
        u = _carry(t, A_carry, W1_l0, W2_l0, W1_carry, W2_carry, active)
        u = _expert(u.reshape(_BS, _D), W1_final, W2_final, active)
        u = _mha(u.reshape(_B, _S, _D), A_cmp)
        u = _expert(u.reshape(_BS, _D), W1_post, W2_post, active)
        return u.reshape(_B, _S, _D)

    def other_path(t):
        u = _expert(t.reshape(_BS, _D), W1_l0, W2_l0, active)
        u3 = u.reshape(_B, _S, _D)
        u3 = jax.lax.cond(
            a0 == 3,
            lambda v: _stack(v.reshape(_BS, _D), W1_div,
                             W2_div).reshape(_B, _S, _D),
            lambda v: v, u3)
        u3 = jax.lax.cond(
            a0 == 4,
            lambda v: _stack(v.reshape(_BS, _D), W1_mod,
                             W2_mod).reshape(_B, _S, _D),
            lambda v: v, u3)
        u3 = jax.lax.cond(
            a0 == 14, lambda v: _mha(v, A_shl),
            lambda v: jax.lax.cond(a0 == 15, lambda w: _mha(w, A_shr),
                                   lambda w: w, v),
            u3)
        u3 = jax.lax.cond(
            a0 == 8, lambda v: _mha(v, A_eq),
            lambda v: jax.lax.cond(a0 == 9, lambda w: _mha(w, A_ne),
                                   lambda w: w, v),
            u3)
        u = _expert(u3.reshape(_BS, _D), W1_final, W2_final, active)
        u = _expert(u, W1_post, W2_post, active)
        return u.reshape(_B, _S, _D)

    return jax.lax.cond(
        a0 <= 2, carry_fused_path,
        lambda t: jax.lax.cond((a0 >= 10) & (a0 <= 13),
                               carry_cmp_path, other_path, t),
        x)
